# baseline (speedup 1.0000x reference)
import functools

import jax
import jax.numpy as jnp
from jax import lax
from jax.experimental import pallas as pl
from jax.experimental.pallas import tpu as pltpu
from jax.experimental.pallas import tpu_sc as plsc

N = 10000
E = 320000
D = 128
H = 128

NC = 2
NS = 16
DH = D // NC

NP = 10240
K = 128
EP = 327680
CW = EP // (NS * K)
SB = 32
G = 2
RPT = NP // NS
CL = 16

_mesh = plsc.VectorSubcoreMesh(
    core_axis_name="c", subcore_axis_name="s", num_cores=NC, num_subcores=NS
)


@functools.partial(
    pl.kernel,
    out_type=[
        jax.ShapeDtypeStruct((NC, NP, DH), jnp.float32),
        jax.ShapeDtypeStruct((NC, NP, CL), jnp.float32),
    ],
    mesh=_mesh,
    compiler_params=pltpu.CompilerParams(use_tc_tiling_on_sc=False),
    scratch_types=[
        pltpu.VMEM((SB, K), jnp.int32),
        pltpu.VMEM((SB, K), jnp.int32),
        pltpu.VMEM((SB, K), jnp.int32),
        pltpu.VMEM((G, K, DH), jnp.float32),
        pltpu.VMEM((K, CL), jnp.float32),
        pltpu.VMEM_SHARED((NP, DH), jnp.float32),
        pltpu.VMEM_SHARED((NP, DH), jnp.float32),
        pltpu.VMEM_SHARED((NP, CL), jnp.float32),
        pltpu.SemaphoreType.DMA((G,)),
        pltpu.SemaphoreType.DMA((G,)),
        pltpu.SemaphoreType.DMA,
    ],
)
def _sc_aggregate(x2_hbm, src_hbm, src2_hbm, dst_hbm,
                  zeros_hbm, zeros16_hbm, ones16_hbm, sidx_hbm,
                  summed_hbm, cnt_hbm,
                  src_v, src2_v, dst_v, rows_v, ones_v, xc, acc, cacc,
                  gsem, ssem, csem):
    c = lax.axis_index("c")
    s = lax.axis_index("s")

    pltpu.sync_copy(sidx_hbm.at[c, pl.ds(s * 5, 5)], src_v.at[pl.ds(0, 5)])
    pltpu.sync_copy(zeros_hbm, acc.at[pl.ds(s * RPT, RPT)])
    pltpu.sync_copy(zeros16_hbm, cacc.at[pl.ds(s * RPT, RPT)])
    pltpu.sync_copy(ones16_hbm, ones_v)
    for j in range(5):
        b = j % G
        pltpu.async_copy(x2_hbm.at[src_v.at[j]], rows_v.at[b],
                         gsem.at[b]).wait()
        pltpu.sync_copy(rows_v.at[b, pl.ds(0, 125)],
                        xc.at[pl.ds(s * 625 + j * 125, 125)])

    plsc.subcore_barrier()

    def _stage_body(t, _):
        base = s * CW + t * SB
        pltpu.sync_copy(src_hbm.at[pl.ds(base, SB)], src_v)
        pltpu.sync_copy(src2_hbm.at[c, pl.ds(base, SB)], src2_v)
        pltpu.sync_copy(dst_hbm.at[pl.ds(base, SB)], dst_v)

        def _group_body(q, _):
            gds = []
            for b in range(G):
                i = G * q + b

                @pl.when(q > 0)
                def _():
                    pltpu.make_async_copy(rows_v.at[b], acc.at[dst_v.at[i]],
                                          ssem.at[b]).wait()
                if b == 0:
                    gds.append(pltpu.async_copy(x2_hbm.at[src2_v.at[i]],
                                                rows_v.at[b], gsem.at[b]))
                else:
                    gds.append(pltpu.async_copy(xc.at[src_v.at[i]],
                                                rows_v.at[b], gsem.at[b]))
            for b in range(G):
                i = G * q + b
                gds[b].wait()
                pltpu.async_copy(rows_v.at[b], acc.at[dst_v.at[i]],
                                 ssem.at[b], add=True)

                @pl.when(c == b)
                def _():
                    pltpu.async_copy(ones_v, cacc.at[dst_v.at[i]],
                                     csem, add=True)
            return ()
        lax.fori_loop(0, SB // G, _group_body, ())

        for b in range(G):
            pltpu.make_async_copy(rows_v.at[b], acc.at[dst_v.at[b]],
                                  ssem.at[b]).wait()

        def _cnt_drain(i, _):
            pltpu.make_async_copy(ones_v, cacc.at[dst_v.at[0]], csem).wait()
            return ()
        lax.fori_loop(0, SB // G, _cnt_drain, ())
        return ()
    lax.fori_loop(0, CW // SB, _stage_body, ())

    plsc.subcore_barrier()

    pltpu.sync_copy(acc.at[pl.ds(s * RPT, RPT)],
                    summed_hbm.at[c, pl.ds(s * RPT, RPT)])
    pltpu.sync_copy(cacc.at[pl.ds(s * RPT, RPT)],
                    cnt_hbm.at[c, pl.ds(s * RPT, RPT)])


_BLK = 200


def _tc_root_body(x_ref, wr_ref, bl_ref, r_ref):
    dn = (((1,), (1,)), ((), ()))
    r_ref[...] = (
        lax.dot_general(x_ref[...], wr_ref[...], dn,
                        preferred_element_type=jnp.float32)
        + bl_ref[...]
    )


_tc_root = pl.pallas_call(
    _tc_root_body,
    out_shape=jax.ShapeDtypeStruct((N, H), jnp.float32),
    grid=(N // _BLK,),
    in_specs=[
        pl.BlockSpec((_BLK, D), lambda i: (i, 0)),
        pl.BlockSpec((H, D), lambda i: (0, 0)),
        pl.BlockSpec((1, H), lambda i: (0, 0)),
    ],
    out_specs=pl.BlockSpec((_BLK, H), lambda i: (i, 0)),
)


def _tc_body(sum_ref, cnt_ref, r_ref, wl_ref, out_ref):
    ssum = jnp.concatenate([sum_ref[0], sum_ref[1]], axis=1)
    cnt = cnt_ref[0, :, 0] + cnt_ref[1, :, 0]
    agg = ssum / jnp.maximum(cnt, 1.0)[:, None]
    dn = (((1,), (1,)), ((), ()))
    out = (
        lax.dot_general(agg, wl_ref[...], dn, preferred_element_type=jnp.float32)
        + r_ref[...]
    )
    norm = jnp.sqrt(jnp.sum(out * out, axis=1, keepdims=True))
    out = out / jnp.maximum(norm, 1e-12)
    out_ref[...] = jnp.maximum(out, 0.0)


_tc_combine = pl.pallas_call(
    _tc_body,
    out_shape=jax.ShapeDtypeStruct((N, H), jnp.float32),
    grid=(N // _BLK,),
    in_specs=[
        pl.BlockSpec((NC, _BLK, DH), lambda i: (0, i, 0)),
        pl.BlockSpec((NC, _BLK, CL), lambda i: (0, i, 0)),
        pl.BlockSpec((_BLK, H), lambda i: (i, 0)),
        pl.BlockSpec((H, D), lambda i: (0, 0)),
    ],
    out_specs=pl.BlockSpec((_BLK, H), lambda i: (i, 0)),
)


def kernel(x, edge_index, W_l, b_l, W_r):
    src = edge_index[0]
    dst = edge_index[1]
    pad = EP - E
    src_p = jnp.concatenate([src, jnp.zeros((pad,), jnp.int32)])
    dst_p = jnp.concatenate([dst, jnp.full((pad,), NP - 1, jnp.int32)])
    srcK = src_p.reshape(EP // K, K)
    dstK = dst_p.reshape(EP // K, K)
    zeros = jnp.zeros((RPT, DH), jnp.float32)
    zeros16 = jnp.zeros((RPT, CL), jnp.float32)
    ones16 = jnp.ones((K, CL), jnp.float32)

    src2 = jnp.stack([2 * src_p, 2 * src_p + 1]).reshape(NC, EP // K, K)
    x2 = lax.optimization_barrier(x.reshape(2 * N, DH))
    kk = jnp.minimum(jnp.arange(K, dtype=jnp.int32), 124)
    rows = jnp.arange(NS * 5, dtype=jnp.int32)[:, None] * 125 + kk[None, :]
    sidx = jnp.stack([2 * rows, 2 * rows + 1])
    summed, cnt = _sc_aggregate(x2, srcK, src2, dstK, zeros, zeros16, ones16,
                                sidx)
    r = _tc_root(x, W_r, b_l.reshape(1, H))
    return _tc_combine(summed, cnt, r, W_l)


# speedup vs baseline: 1.2827x; 1.2827x over previous
import functools

import jax
import jax.numpy as jnp
from jax import lax
from jax.experimental import pallas as pl
from jax.experimental.pallas import tpu as pltpu
from jax.experimental.pallas import tpu_sc as plsc

N = 10000
E = 320000
D = 128
H = 128

NC = 2
NS = 16
DH = D // NC

NP = 10240
K = 128
EP = 327680
CW = EP // (NS * K)
SB = 32
G = 2
RPT = NP // NS
CL = 16

_mesh = plsc.VectorSubcoreMesh(
    core_axis_name="c", subcore_axis_name="s", num_cores=NC, num_subcores=NS
)


@functools.partial(
    pl.kernel,
    out_type=[
        jax.ShapeDtypeStruct((NC, NP, DH), jnp.float32),
        jax.ShapeDtypeStruct((NC, NP, CL), jnp.float32),
    ],
    mesh=_mesh,
    compiler_params=pltpu.CompilerParams(use_tc_tiling_on_sc=False),
    scratch_types=[
        pltpu.VMEM((SB, K), jnp.int32),
        pltpu.VMEM((SB, K), jnp.int32),
        pltpu.VMEM((G, K, DH), jnp.float32),
        pltpu.VMEM((K, CL), jnp.float32),
        pltpu.VMEM_SHARED((NP, DH), jnp.float32),
        pltpu.VMEM_SHARED((NP, DH), jnp.float32),
        pltpu.VMEM_SHARED((NP, CL), jnp.float32),
        pltpu.SemaphoreType.DMA((G,)),
        pltpu.SemaphoreType.DMA((G,)),
        pltpu.SemaphoreType.DMA,
    ],
)
def _sc_aggregate(x_hbm, src_hbm, dst_hbm, zeros_hbm, zeros16_hbm, ones16_hbm,
                  summed_hbm, cnt_hbm,
                  src_v, dst_v, rows_v, ones_v, xc, acc, cacc, gsem, ssem, csem):
    c = lax.axis_index("c")
    s = lax.axis_index("s")

    pltpu.sync_copy(x_hbm.at[pl.ds(s * (N // NS), N // NS), pl.ds(c * DH, DH)],
                    xc.at[pl.ds(s * (N // NS), N // NS)])
    pltpu.sync_copy(zeros_hbm, acc.at[pl.ds(s * RPT, RPT)])
    pltpu.sync_copy(zeros16_hbm, cacc.at[pl.ds(s * RPT, RPT)])
    pltpu.sync_copy(ones16_hbm, ones_v)

    plsc.subcore_barrier()

    def _stage_body(t, _):
        base = s * CW + t * SB
        pltpu.sync_copy(src_hbm.at[pl.ds(base, SB)], src_v)
        pltpu.sync_copy(dst_hbm.at[pl.ds(base, SB)], dst_v)

        def _group_body(q, _):
            gds = []
            for b in range(G):
                i = G * q + b

                @pl.when(q > 0)
                def _():
                    pltpu.make_async_copy(rows_v.at[b], acc.at[dst_v.at[i]],
                                          ssem.at[b]).wait()
                gds.append(pltpu.async_copy(xc.at[src_v.at[i]], rows_v.at[b],
                                            gsem.at[b]))
            for b in range(G):
                i = G * q + b
                gds[b].wait()
                pltpu.async_copy(rows_v.at[b], acc.at[dst_v.at[i]],
                                 ssem.at[b], add=True)

                @pl.when(c == b)
                def _():
                    pltpu.async_copy(ones_v, cacc.at[dst_v.at[i]],
                                     csem, add=True)
            return ()
        lax.fori_loop(0, SB // G, _group_body, ())

        for b in range(G):
            pltpu.make_async_copy(rows_v.at[b], acc.at[dst_v.at[b]],
                                  ssem.at[b]).wait()

        def _cnt_drain(i, _):
            pltpu.make_async_copy(ones_v, cacc.at[dst_v.at[0]], csem).wait()
            return ()
        lax.fori_loop(0, SB // G, _cnt_drain, ())
        return ()
    lax.fori_loop(0, CW // SB, _stage_body, ())

    plsc.subcore_barrier()

    pltpu.sync_copy(acc.at[pl.ds(s * RPT, RPT)],
                    summed_hbm.at[c, pl.ds(s * RPT, RPT)])
    pltpu.sync_copy(cacc.at[pl.ds(s * RPT, RPT)],
                    cnt_hbm.at[c, pl.ds(s * RPT, RPT)])


_BLK = 200


def _tc_root_body(x_ref, wr_ref, bl_ref, r_ref):
    dn = (((1,), (1,)), ((), ()))
    r_ref[...] = (
        lax.dot_general(x_ref[...], wr_ref[...], dn,
                        preferred_element_type=jnp.float32)
        + bl_ref[...]
    )


_tc_root = pl.pallas_call(
    _tc_root_body,
    out_shape=jax.ShapeDtypeStruct((N, H), jnp.float32),
    grid=(N // _BLK,),
    in_specs=[
        pl.BlockSpec((_BLK, D), lambda i: (i, 0)),
        pl.BlockSpec((H, D), lambda i: (0, 0)),
        pl.BlockSpec((1, H), lambda i: (0, 0)),
    ],
    out_specs=pl.BlockSpec((_BLK, H), lambda i: (i, 0)),
)


def _tc_body(sum_ref, cnt_ref, r_ref, wl_ref, out_ref):
    ssum = jnp.concatenate([sum_ref[0], sum_ref[1]], axis=1)
    cnt = cnt_ref[0, :, 0] + cnt_ref[1, :, 0]
    agg = ssum / jnp.maximum(cnt, 1.0)[:, None]
    dn = (((1,), (1,)), ((), ()))
    out = (
        lax.dot_general(agg, wl_ref[...], dn, preferred_element_type=jnp.float32)
        + r_ref[...]
    )
    norm = jnp.sqrt(jnp.sum(out * out, axis=1, keepdims=True))
    out = out / jnp.maximum(norm, 1e-12)
    out_ref[...] = jnp.maximum(out, 0.0)


_tc_combine = pl.pallas_call(
    _tc_body,
    out_shape=jax.ShapeDtypeStruct((N, H), jnp.float32),
    grid=(N // _BLK,),
    in_specs=[
        pl.BlockSpec((NC, _BLK, DH), lambda i: (0, i, 0)),
        pl.BlockSpec((NC, _BLK, CL), lambda i: (0, i, 0)),
        pl.BlockSpec((_BLK, H), lambda i: (i, 0)),
        pl.BlockSpec((H, D), lambda i: (0, 0)),
    ],
    out_specs=pl.BlockSpec((_BLK, H), lambda i: (i, 0)),
)


def kernel(x, edge_index, W_l, b_l, W_r):
    src = edge_index[0]
    dst = edge_index[1]
    pad = EP - E
    src_p = jnp.concatenate([src, jnp.zeros((pad,), jnp.int32)])
    dst_p = jnp.concatenate([dst, jnp.full((pad,), NP - 1, jnp.int32)])
    srcK = src_p.reshape(EP // K, K)
    dstK = dst_p.reshape(EP // K, K)
    zeros = jnp.zeros((RPT, DH), jnp.float32)
    zeros16 = jnp.zeros((RPT, CL), jnp.float32)
    ones16 = jnp.ones((K, CL), jnp.float32)

    summed, cnt = _sc_aggregate(x, srcK, dstK, zeros, zeros16, ones16)
    r = _tc_root(x, W_r, b_l.reshape(1, H))
    return _tc_combine(summed, cnt, r, W_l)


# speedup vs baseline: 1.4978x; 1.1678x over previous
import functools

import jax
import jax.numpy as jnp
from jax import lax
from jax.experimental import pallas as pl
from jax.experimental.pallas import tpu as pltpu
from jax.experimental.pallas import tpu_sc as plsc

N = 10000
E = 320000
D = 128
H = 128

NC = 2
NS = 16
DH = D // NC

NP = 10240
K = 128
EP = 327680
CW = EP // (NS * K)
SB = 64
G = 2
RPT = NP // NS
CL = 16

_mesh = plsc.VectorSubcoreMesh(
    core_axis_name="c", subcore_axis_name="s", num_cores=NC, num_subcores=NS
)


@functools.partial(
    pl.kernel,
    out_type=[
        jax.ShapeDtypeStruct((NC, NP, DH), jnp.float32),
        jax.ShapeDtypeStruct((NC, NP, CL), jnp.float32),
    ],
    mesh=_mesh,
    compiler_params=pltpu.CompilerParams(use_tc_tiling_on_sc=False),
    scratch_types=[
        pltpu.VMEM((SB, K), jnp.int32),
        pltpu.VMEM((SB, K), jnp.int32),
        pltpu.VMEM((G, K, DH), jnp.float32),
        pltpu.VMEM((K, CL), jnp.float32),
        pltpu.VMEM_SHARED((NP, DH), jnp.float32),
        pltpu.VMEM_SHARED((NP, DH), jnp.float32),
        pltpu.VMEM_SHARED((NP, CL), jnp.float32),
        pltpu.SemaphoreType.DMA((G,)),
        pltpu.SemaphoreType.DMA((G,)),
        pltpu.SemaphoreType.DMA,
    ],
)
def _sc_aggregate(x_hbm, src_hbm, dst_hbm, zeros_hbm, zeros16_hbm, ones16_hbm,
                  summed_hbm, cnt_hbm,
                  src_v, dst_v, rows_v, ones_v, xc, acc, cacc, gsem, ssem, csem):
    c = lax.axis_index("c")
    s = lax.axis_index("s")

    pltpu.sync_copy(x_hbm.at[pl.ds(s * (N // NS), N // NS), pl.ds(c * DH, DH)],
                    xc.at[pl.ds(s * (N // NS), N // NS)])
    pltpu.sync_copy(zeros_hbm, acc.at[pl.ds(s * RPT, RPT)])
    pltpu.sync_copy(zeros16_hbm, cacc.at[pl.ds(s * RPT, RPT)])
    pltpu.sync_copy(ones16_hbm, ones_v)

    plsc.subcore_barrier()

    def _stage_body(t, _):
        base = s * CW + t * SB
        pltpu.sync_copy(src_hbm.at[pl.ds(base, SB)], src_v)
        pltpu.sync_copy(dst_hbm.at[pl.ds(base, SB)], dst_v)

        def _group_body(q, _):
            gds = []
            for b in range(G):
                i = G * q + b

                @pl.when(q > 0)
                def _():
                    pltpu.make_async_copy(rows_v.at[b], acc.at[dst_v.at[i]],
                                          ssem.at[b]).wait()
                gds.append(pltpu.async_copy(xc.at[src_v.at[i]], rows_v.at[b],
                                            gsem.at[b]))
            for b in range(G):
                i = G * q + b
                gds[b].wait()
                pltpu.async_copy(rows_v.at[b], acc.at[dst_v.at[i]],
                                 ssem.at[b], add=True)

                @pl.when(c == b)
                def _():
                    pltpu.async_copy(ones_v, cacc.at[dst_v.at[i]],
                                     csem, add=True)
            return ()
        lax.fori_loop(0, SB // G, _group_body, ())

        for b in range(G):
            pltpu.make_async_copy(rows_v.at[b], acc.at[dst_v.at[b]],
                                  ssem.at[b]).wait()

        def _cnt_drain(i, _):
            pltpu.make_async_copy(ones_v, cacc.at[dst_v.at[0]], csem).wait()
            return ()
        lax.fori_loop(0, SB // G, _cnt_drain, ())
        return ()
    lax.fori_loop(0, CW // SB, _stage_body, ())

    plsc.subcore_barrier()

    pltpu.sync_copy(acc.at[pl.ds(s * RPT, RPT)],
                    summed_hbm.at[c, pl.ds(s * RPT, RPT)])
    pltpu.sync_copy(cacc.at[pl.ds(s * RPT, RPT)],
                    cnt_hbm.at[c, pl.ds(s * RPT, RPT)])


_BLK = 200


def _tc_root_body(x_ref, wr_ref, bl_ref, r_ref):
    dn = (((1,), (1,)), ((), ()))
    r_ref[...] = (
        lax.dot_general(x_ref[...], wr_ref[...], dn,
                        preferred_element_type=jnp.float32)
        + bl_ref[...]
    )


_tc_root = pl.pallas_call(
    _tc_root_body,
    out_shape=jax.ShapeDtypeStruct((N, H), jnp.float32),
    grid=(N // _BLK,),
    in_specs=[
        pl.BlockSpec((_BLK, D), lambda i: (i, 0)),
        pl.BlockSpec((H, D), lambda i: (0, 0)),
        pl.BlockSpec((1, H), lambda i: (0, 0)),
    ],
    out_specs=pl.BlockSpec((_BLK, H), lambda i: (i, 0)),
)


def _tc_body(sum_ref, cnt_ref, r_ref, wl_ref, out_ref):
    ssum = jnp.concatenate([sum_ref[0], sum_ref[1]], axis=1)
    cnt = cnt_ref[0, :, 0] + cnt_ref[1, :, 0]
    agg = ssum / jnp.maximum(cnt, 1.0)[:, None]
    dn = (((1,), (1,)), ((), ()))
    out = (
        lax.dot_general(agg, wl_ref[...], dn, preferred_element_type=jnp.float32)
        + r_ref[...]
    )
    norm = jnp.sqrt(jnp.sum(out * out, axis=1, keepdims=True))
    out = out / jnp.maximum(norm, 1e-12)
    out_ref[...] = jnp.maximum(out, 0.0)


_tc_combine = pl.pallas_call(
    _tc_body,
    out_shape=jax.ShapeDtypeStruct((N, H), jnp.float32),
    grid=(N // _BLK,),
    in_specs=[
        pl.BlockSpec((NC, _BLK, DH), lambda i: (0, i, 0)),
        pl.BlockSpec((NC, _BLK, CL), lambda i: (0, i, 0)),
        pl.BlockSpec((_BLK, H), lambda i: (i, 0)),
        pl.BlockSpec((H, D), lambda i: (0, 0)),
    ],
    out_specs=pl.BlockSpec((_BLK, H), lambda i: (i, 0)),
)


def kernel(x, edge_index, W_l, b_l, W_r):
    src = edge_index[0]
    dst = edge_index[1]
    pad = EP - E
    src_p = jnp.concatenate([src, jnp.zeros((pad,), jnp.int32)])
    dst_p = jnp.concatenate([dst, jnp.full((pad,), NP - 1, jnp.int32)])
    srcK = src_p.reshape(EP // K, K)
    dstK = dst_p.reshape(EP // K, K)
    zeros = jnp.zeros((RPT, DH), jnp.float32)
    zeros16 = jnp.zeros((RPT, CL), jnp.float32)
    ones16 = jnp.ones((K, CL), jnp.float32)

    summed, cnt = _sc_aggregate(x, srcK, dstK, zeros, zeros16, ones16)
    r = _tc_root(x, W_r, b_l.reshape(1, H))
    return _tc_combine(summed, cnt, r, W_l)
